# baseline (speedup 1.0000x reference)
import functools

import jax
import jax.numpy as jnp
from jax.experimental import pallas as pl
from jax.experimental.pallas import tpu as pltpu


def _router_body(ui_ref, W1_ref, b1_ref, W2_ref, b2_ref, Wr_ref, br_ref,
                 Wt_ref, bt_ref, Wg1t_ref, bg1_ref, Wo1t_ref, bo1_ref,
                 Wa1t_ref, ba1_ref,
                 w_ref, pg_ref, po_ref, pa_ref):
    h = ui_ref[...]
    W1 = W1_ref[...]
    W2 = W2_ref[...]
    b1 = b1_ref[...]
    b2 = b2_ref[...]
    for _ in range(3):
        m = jnp.tanh(jnp.dot(h, W1, preferred_element_type=jnp.float32) + b1)
        h = h + jnp.tanh(jnp.dot(m, W2, preferred_element_type=jnp.float32) + b2)
    logits = jnp.dot(h, Wr_ref[...], preferred_element_type=jnp.float32) + br_ref[...]
    logits = logits - jnp.max(logits, axis=-1, keepdims=True)
    e = jnp.exp(logits)
    w_ref[...] = e / jnp.sum(e, axis=-1, keepdims=True)
    th = jnp.tanh(jnp.dot(h, Wt_ref[...], preferred_element_type=jnp.float32) + bt_ref[...])
    pg_ref[...] = jnp.dot(th, Wg1t_ref[...], preferred_element_type=jnp.float32) + bg1_ref[...]
    po_ref[...] = jnp.dot(th, Wo1t_ref[...], preferred_element_type=jnp.float32) + bo1_ref[...]
    pa_ref[...] = jnp.dot(th, Wa1t_ref[...], preferred_element_type=jnp.float32) + ba1_ref[...]


def _experts_body(TN, H, c3_ref, c2_ref, c1_ref, A1_ref,
                  Wg2_ref, bg2_ref, Wo2_ref, bo2_ref, Wa2_ref, ba2_ref,
                  Whead_ref, bhead_ref, w_ref,
                  sdf_ref, img_ref, aud_ref):
    w = w_ref[0]
    bf16 = jnp.bfloat16
    ones = jnp.ones((TN, 1), bf16)
    zeros = jnp.zeros((TN, 1), bf16)
    cat = jnp.concatenate(
        [c3_ref[0].astype(bf16), c2_ref[0].astype(bf16),
         c1_ref[0].astype(bf16), ones, zeros], axis=-1)
    h1 = jnp.dot(cat, A1_ref[0], preferred_element_type=jnp.float32)
    h1 = jnp.maximum(h1, 0.0).astype(bf16)

    hg = jnp.dot(h1[:, :H], Wg2_ref[...],
                 preferred_element_type=jnp.float32) + bg2_ref[...]
    ho = jnp.dot(h1[:, H:2 * H], Wo2_ref[...],
                 preferred_element_type=jnp.float32) + bo2_ref[...]
    ha = jnp.dot(h1[:, 2 * H:], Wa2_ref[...],
                 preferred_element_type=jnp.float32) + ba2_ref[...]
    h2 = jnp.maximum(jnp.concatenate([hg, ho, ha], axis=0), 0.0).astype(bf16)

    out = jnp.dot(h2, Whead_ref[...],
                  preferred_element_type=jnp.float32) + bhead_ref[...]
    sdf_ref[0] = out[:TN, 0:1] * w[0:1, 0:1]
    img_ref[0] = jax.nn.sigmoid(out[TN:2 * TN, 1:4]) * w[0:1, 1:2]
    aud_ref[0] = jnp.tanh(out[2 * TN:, 4:5]) * w[0:1, 2:3]


@functools.partial(jax.jit, static_argnames=("interpret",))
def kernel(user_intent, coords_3d, coords_2d, coords_1d,
           W1, b1, W2, b2, Wr, br, Wt, bt,
           Wg1, bg1, Wg2, bg2, Wgs, bgs, Wgc, bgc,
           Wo1, bo1, Wo2, bo2, Wo3, bo3,
           Wa1, ba1, Wa2, ba2, Wa3, ba3, interpret=False):
    B, N, _ = coords_3d.shape
    GD = user_intent.shape[1]
    H = Wg2.shape[0]
    TN = 512
    f32 = jnp.float32

    w, pg, po, pa = pl.pallas_call(
        _router_body,
        out_shape=(
            jax.ShapeDtypeStruct((B, 3), f32),
            jax.ShapeDtypeStruct((B, H), f32),
            jax.ShapeDtypeStruct((B, H), f32),
            jax.ShapeDtypeStruct((B, H), f32),
        ),
        interpret=interpret,
    )(user_intent, W1, b1.reshape(1, GD), W2, b2.reshape(1, GD),
      Wr, br.reshape(1, 3), Wt, bt.reshape(1, -1),
      Wg1[3:], bg1.reshape(1, H), Wo1[2:], bo1.reshape(1, H),
      Wa1[1:], ba1.reshape(1, H))

    z = jnp.zeros((B, 1, H), f32)
    zc = jnp.zeros((B, 8, H), f32)
    bcast = lambda W: jnp.broadcast_to(W[None], (B,) + W.shape)
    Ag = jnp.concatenate([bcast(Wg1[:3]), z, z, z, pg[:, None, :], z], axis=1)
    Ao = jnp.concatenate([z, z, z, bcast(Wo1[:2]), z, po[:, None, :], z],
                         axis=1)
    Aa = jnp.concatenate([z, z, z, z, z, bcast(Wa1[:1]), pa[:, None, :], z],
                         axis=1)
    A1 = jnp.concatenate([Ag, Ao, Aa], axis=2).astype(jnp.bfloat16)

    Whead = jnp.concatenate([Wgs, Wo3, Wa3, jnp.zeros((H, 3), f32)],
                            axis=1).astype(jnp.bfloat16)
    bhead = jnp.concatenate([bgs, bo3, ba3, jnp.zeros((3,), f32)]).reshape(1, 8)

    def const(shape):
        return pl.BlockSpec(shape, lambda b, n: (0, 0))

    sdf, img, aud = pl.pallas_call(
        functools.partial(_experts_body, TN, H),
        grid=(B, N // TN),
        in_specs=[
            pl.BlockSpec((1, TN, 3), lambda b, n: (b, n, 0)),
            pl.BlockSpec((1, TN, 2), lambda b, n: (b, n, 0)),
            pl.BlockSpec((1, TN, 1), lambda b, n: (b, n, 0)),
            pl.BlockSpec((1, 8, 3 * H), lambda b, n: (b, 0, 0)),
            const((H, H)), const((1, H)),
            const((H, H)), const((1, H)),
            const((H, H)), const((1, H)),
            const((H, 8)), const((1, 8)),
            pl.BlockSpec((1, 1, 3), lambda b, n: (b, 0, 0)),
        ],
        out_specs=[
            pl.BlockSpec((1, TN, 1), lambda b, n: (b, n, 0)),
            pl.BlockSpec((1, TN, 3), lambda b, n: (b, n, 0)),
            pl.BlockSpec((1, TN, 1), lambda b, n: (b, n, 0)),
        ],
        out_shape=(
            jax.ShapeDtypeStruct((B, N, 1), f32),
            jax.ShapeDtypeStruct((B, N, 3), f32),
            jax.ShapeDtypeStruct((B, N, 1), f32),
        ),
        compiler_params=pltpu.CompilerParams(
            dimension_semantics=("parallel", "parallel")),
        interpret=interpret,
    )(coords_3d, coords_2d, coords_1d, A1,
      Wg2.astype(jnp.bfloat16), bg2.reshape(1, H),
      Wo2.astype(jnp.bfloat16), bo2.reshape(1, H),
      Wa2.astype(jnp.bfloat16), ba2.reshape(1, H),
      Whead, bhead, w.reshape(B, 1, 3))

    return (w, sdf, img, aud)


# speedup vs baseline: 1.1803x; 1.1803x over previous
import functools

import jax
import jax.numpy as jnp
from jax.experimental import pallas as pl
from jax.experimental.pallas import tpu as pltpu


def _router_body(ui_ref, W1_ref, b1_ref, W2_ref, b2_ref, Wr_ref, br_ref,
                 Wt_ref, bt_ref, Wg1t_ref, bg1_ref, Wo1t_ref, bo1_ref,
                 Wa1t_ref, ba1_ref,
                 w_ref, pg_ref, po_ref, pa_ref):
    h = ui_ref[...]
    W1 = W1_ref[...]
    W2 = W2_ref[...]
    b1 = b1_ref[...]
    b2 = b2_ref[...]
    for _ in range(3):
        m = jnp.tanh(jnp.dot(h, W1, preferred_element_type=jnp.float32) + b1)
        h = h + jnp.tanh(jnp.dot(m, W2, preferred_element_type=jnp.float32) + b2)
    logits = jnp.dot(h, Wr_ref[...], preferred_element_type=jnp.float32) + br_ref[...]
    logits = logits - jnp.max(logits, axis=-1, keepdims=True)
    e = jnp.exp(logits)
    w_ref[...] = e / jnp.sum(e, axis=-1, keepdims=True)
    th = jnp.tanh(jnp.dot(h, Wt_ref[...], preferred_element_type=jnp.float32) + bt_ref[...])
    pg_ref[...] = jnp.dot(th, Wg1t_ref[...], preferred_element_type=jnp.float32) + bg1_ref[...]
    po_ref[...] = jnp.dot(th, Wo1t_ref[...], preferred_element_type=jnp.float32) + bo1_ref[...]
    pa_ref[...] = jnp.dot(th, Wa1t_ref[...], preferred_element_type=jnp.float32) + ba1_ref[...]


def _experts_body(TN, H, c3_ref, c2_ref, c1_ref, A1_ref,
                  Wg2_ref, bg2_ref, Wo2_ref, bo2_ref, Wa2_ref, ba2_ref,
                  Whead_ref, bhead_ref, w_ref,
                  sdf_ref, img_ref, aud_ref):
    w = w_ref[0]
    bf16 = jnp.bfloat16
    ones = jnp.ones((TN, 1), bf16)
    zeros = jnp.zeros((TN, 1), bf16)
    cat = jnp.concatenate(
        [c3_ref[0].astype(bf16), c2_ref[0].astype(bf16),
         c1_ref[0].astype(bf16), ones, zeros], axis=-1)
    h1 = jnp.dot(cat, A1_ref[0], preferred_element_type=jnp.float32)
    h1 = jnp.maximum(h1, 0.0).astype(bf16)

    hg = jnp.dot(h1[:, :H], Wg2_ref[...],
                 preferred_element_type=jnp.float32) + bg2_ref[...]
    ho = jnp.dot(h1[:, H:2 * H], Wo2_ref[...],
                 preferred_element_type=jnp.float32) + bo2_ref[...]
    ha = jnp.dot(h1[:, 2 * H:], Wa2_ref[...],
                 preferred_element_type=jnp.float32) + ba2_ref[...]
    h2 = jnp.maximum(jnp.concatenate([hg, ho, ha], axis=0), 0.0).astype(bf16)

    out = jnp.dot(h2, Whead_ref[...],
                  preferred_element_type=jnp.float32) + bhead_ref[...]
    sdf_ref[0] = out[:TN, 0:1] * w[0:1, 0:1]
    img_ref[0] = jax.nn.sigmoid(out[TN:2 * TN, 1:4]) * w[0:1, 1:2]
    aud_ref[0] = jnp.tanh(out[2 * TN:, 4:5]) * w[0:1, 2:3]


@functools.partial(jax.jit, static_argnames=("interpret",))
def kernel(user_intent, coords_3d, coords_2d, coords_1d,
           W1, b1, W2, b2, Wr, br, Wt, bt,
           Wg1, bg1, Wg2, bg2, Wgs, bgs, Wgc, bgc,
           Wo1, bo1, Wo2, bo2, Wo3, bo3,
           Wa1, ba1, Wa2, ba2, Wa3, ba3, interpret=False):
    B, N, _ = coords_3d.shape
    GD = user_intent.shape[1]
    H = Wg2.shape[0]
    TN = 1024
    f32 = jnp.float32

    w, pg, po, pa = pl.pallas_call(
        _router_body,
        out_shape=(
            jax.ShapeDtypeStruct((B, 3), f32),
            jax.ShapeDtypeStruct((B, H), f32),
            jax.ShapeDtypeStruct((B, H), f32),
            jax.ShapeDtypeStruct((B, H), f32),
        ),
        interpret=interpret,
    )(user_intent, W1, b1.reshape(1, GD), W2, b2.reshape(1, GD),
      Wr, br.reshape(1, 3), Wt, bt.reshape(1, -1),
      Wg1[3:], bg1.reshape(1, H), Wo1[2:], bo1.reshape(1, H),
      Wa1[1:], ba1.reshape(1, H))

    z = jnp.zeros((B, 1, H), f32)
    zc = jnp.zeros((B, 8, H), f32)
    bcast = lambda W: jnp.broadcast_to(W[None], (B,) + W.shape)
    Ag = jnp.concatenate([bcast(Wg1[:3]), z, z, z, pg[:, None, :], z], axis=1)
    Ao = jnp.concatenate([z, z, z, bcast(Wo1[:2]), z, po[:, None, :], z],
                         axis=1)
    Aa = jnp.concatenate([z, z, z, z, z, bcast(Wa1[:1]), pa[:, None, :], z],
                         axis=1)
    A1 = jnp.concatenate([Ag, Ao, Aa], axis=2).astype(jnp.bfloat16)

    Whead = jnp.concatenate([Wgs, Wo3, Wa3, jnp.zeros((H, 3), f32)],
                            axis=1).astype(jnp.bfloat16)
    bhead = jnp.concatenate([bgs, bo3, ba3, jnp.zeros((3,), f32)]).reshape(1, 8)

    def const(shape):
        return pl.BlockSpec(shape, lambda b, n: (0, 0))

    sdf, img, aud = pl.pallas_call(
        functools.partial(_experts_body, TN, H),
        grid=(B, N // TN),
        in_specs=[
            pl.BlockSpec((1, TN, 3), lambda b, n: (b, n, 0)),
            pl.BlockSpec((1, TN, 2), lambda b, n: (b, n, 0)),
            pl.BlockSpec((1, TN, 1), lambda b, n: (b, n, 0)),
            pl.BlockSpec((1, 8, 3 * H), lambda b, n: (b, 0, 0)),
            const((H, H)), const((1, H)),
            const((H, H)), const((1, H)),
            const((H, H)), const((1, H)),
            const((H, 8)), const((1, 8)),
            pl.BlockSpec((1, 1, 3), lambda b, n: (b, 0, 0)),
        ],
        out_specs=[
            pl.BlockSpec((1, TN, 1), lambda b, n: (b, n, 0)),
            pl.BlockSpec((1, TN, 3), lambda b, n: (b, n, 0)),
            pl.BlockSpec((1, TN, 1), lambda b, n: (b, n, 0)),
        ],
        out_shape=(
            jax.ShapeDtypeStruct((B, N, 1), f32),
            jax.ShapeDtypeStruct((B, N, 3), f32),
            jax.ShapeDtypeStruct((B, N, 1), f32),
        ),
        compiler_params=pltpu.CompilerParams(
            dimension_semantics=("parallel", "parallel")),
        interpret=interpret,
    )(coords_3d, coords_2d, coords_1d, A1,
      Wg2.astype(jnp.bfloat16), bg2.reshape(1, H),
      Wo2.astype(jnp.bfloat16), bo2.reshape(1, H),
      Wa2.astype(jnp.bfloat16), ba2.reshape(1, H),
      Whead, bhead, w.reshape(B, 1, 3))

    return (w, sdf, img, aud)


# speedup vs baseline: 1.2344x; 1.0458x over previous
import functools

import jax
import jax.numpy as jnp
from jax.experimental import pallas as pl
from jax.experimental.pallas import tpu as pltpu


def _router_body(ui_ref, W1_ref, b1_ref, W2_ref, b2_ref, Wr_ref, br_ref,
                 Wt_ref, bt_ref, Wg1_ref, bg1_ref, Wo1_ref, bo1_ref,
                 Wa1_ref, ba1_ref,
                 w_ref, bias_ref, wf_ref):
    h = ui_ref[...]
    W1 = W1_ref[...]
    W2 = W2_ref[...]
    b1 = b1_ref[...]
    b2 = b2_ref[...]
    for _ in range(3):
        m = jnp.tanh(jnp.dot(h, W1, preferred_element_type=jnp.float32) + b1)
        h = h + jnp.tanh(jnp.dot(m, W2, preferred_element_type=jnp.float32) + b2)
    logits = jnp.dot(h, Wr_ref[...], preferred_element_type=jnp.float32) + br_ref[...]
    logits = logits - jnp.max(logits, axis=-1, keepdims=True)
    e = jnp.exp(logits)
    w_ref[...] = e / jnp.sum(e, axis=-1, keepdims=True)
    th = jnp.tanh(jnp.dot(h, Wt_ref[...], preferred_element_type=jnp.float32) + bt_ref[...])
    pg = jnp.dot(th, Wg1_ref[3:], preferred_element_type=jnp.float32) + bg1_ref[...]
    po = jnp.dot(th, Wo1_ref[2:], preferred_element_type=jnp.float32) + bo1_ref[...]
    pa = jnp.dot(th, Wa1_ref[1:], preferred_element_type=jnp.float32) + ba1_ref[...]
    bias_ref[...] = jnp.concatenate([pg, po, pa], axis=1)
    H = bg1_ref.shape[1]
    z = lambda r: jnp.zeros((r, H), jnp.float32)
    wg = jnp.concatenate([Wg1_ref[:3], z(5)], axis=0)
    wo = jnp.concatenate([z(3), Wo1_ref[:2], z(3)], axis=0)
    wa = jnp.concatenate([z(5), Wa1_ref[:1], z(2)], axis=0)
    wf_ref[...] = jnp.concatenate([wg, wo, wa], axis=1).astype(jnp.bfloat16)


def _experts_body(TN, H, c3_ref, c2_ref, c1_ref, wf_ref, bias_ref,
                  Wg2_ref, bg2_ref, Wo2_ref, bo2_ref, Wa2_ref, ba2_ref,
                  Wgs_ref, bgs_ref, Wo3_ref, bo3_ref, Wa3_ref, ba3_ref,
                  w_ref,
                  sdf_ref, img_ref, aud_ref):
    w = w_ref[0]
    bf16 = jnp.bfloat16
    pad = jnp.zeros((TN, 2), bf16)
    cat = jnp.concatenate(
        [c3_ref[0].astype(bf16), c2_ref[0].astype(bf16),
         c1_ref[0].astype(bf16), pad], axis=-1)
    h1 = jnp.dot(cat, wf_ref[...], preferred_element_type=jnp.float32)
    h1 = jnp.maximum(h1 + bias_ref[0], 0.0).astype(bf16)

    hg = jnp.dot(h1[:, :H], Wg2_ref[...],
                 preferred_element_type=jnp.float32) + bg2_ref[...]
    ho = jnp.dot(h1[:, H:2 * H], Wo2_ref[...],
                 preferred_element_type=jnp.float32) + bo2_ref[...]
    ha = jnp.dot(h1[:, 2 * H:], Wa2_ref[...],
                 preferred_element_type=jnp.float32) + ba2_ref[...]
    hg = jnp.maximum(hg, 0.0)
    ho = jnp.maximum(ho, 0.0)
    ha = jnp.maximum(ha, 0.0)

    sdf = jnp.dot(hg, Wgs_ref[...], preferred_element_type=jnp.float32)
    img = jnp.dot(ho, Wo3_ref[...], preferred_element_type=jnp.float32)
    aud = jnp.dot(ha, Wa3_ref[...], preferred_element_type=jnp.float32)
    sdf_ref[0] = (sdf + bgs_ref[...]) * w[0:1, 0:1]
    img_ref[0] = jax.nn.sigmoid(img + bo3_ref[...]) * w[0:1, 1:2]
    aud_ref[0] = jnp.tanh(aud + ba3_ref[...]) * w[0:1, 2:3]


@functools.partial(jax.jit, static_argnames=("interpret",))
def kernel(user_intent, coords_3d, coords_2d, coords_1d,
           W1, b1, W2, b2, Wr, br, Wt, bt,
           Wg1, bg1, Wg2, bg2, Wgs, bgs, Wgc, bgc,
           Wo1, bo1, Wo2, bo2, Wo3, bo3,
           Wa1, ba1, Wa2, ba2, Wa3, ba3, interpret=False):
    B, N, _ = coords_3d.shape
    GD = user_intent.shape[1]
    H = Wg2.shape[0]
    TN = 1024
    f32 = jnp.float32
    bf16 = jnp.bfloat16

    w, bias768, Wfirst = pl.pallas_call(
        _router_body,
        out_shape=(
            jax.ShapeDtypeStruct((B, 3), f32),
            jax.ShapeDtypeStruct((B, 3 * H), f32),
            jax.ShapeDtypeStruct((8, 3 * H), bf16),
        ),
        interpret=interpret,
    )(user_intent, W1, b1.reshape(1, GD), W2, b2.reshape(1, GD),
      Wr, br.reshape(1, 3), Wt, bt.reshape(1, -1),
      Wg1, bg1.reshape(1, H), Wo1, bo1.reshape(1, H),
      Wa1, ba1.reshape(1, H))

    def const(shape):
        return pl.BlockSpec(shape, lambda b, n: (0, 0))

    sdf, img, aud = pl.pallas_call(
        functools.partial(_experts_body, TN, H),
        grid=(B, N // TN),
        in_specs=[
            pl.BlockSpec((1, TN, 3), lambda b, n: (b, n, 0)),
            pl.BlockSpec((1, TN, 2), lambda b, n: (b, n, 0)),
            pl.BlockSpec((1, TN, 1), lambda b, n: (b, n, 0)),
            const((8, 3 * H)),
            pl.BlockSpec((1, 1, 3 * H), lambda b, n: (b, 0, 0)),
            const((H, H)), const((1, H)),
            const((H, H)), const((1, H)),
            const((H, H)), const((1, H)),
            const((H, 1)), const((1, 1)),
            const((H, 3)), const((1, 3)),
            const((H, 1)), const((1, 1)),
            pl.BlockSpec((1, 1, 3), lambda b, n: (b, 0, 0)),
        ],
        out_specs=[
            pl.BlockSpec((1, TN, 1), lambda b, n: (b, n, 0)),
            pl.BlockSpec((1, TN, 3), lambda b, n: (b, n, 0)),
            pl.BlockSpec((1, TN, 1), lambda b, n: (b, n, 0)),
        ],
        out_shape=(
            jax.ShapeDtypeStruct((B, N, 1), f32),
            jax.ShapeDtypeStruct((B, N, 3), f32),
            jax.ShapeDtypeStruct((B, N, 1), f32),
        ),
        compiler_params=pltpu.CompilerParams(
            dimension_semantics=("parallel", "parallel")),
        interpret=interpret,
    )(coords_3d, coords_2d, coords_1d, Wfirst, bias768.reshape(B, 1, 3 * H),
      Wg2.astype(bf16), bg2.reshape(1, H),
      Wo2.astype(bf16), bo2.reshape(1, H),
      Wa2.astype(bf16), ba2.reshape(1, H),
      Wgs, bgs.reshape(1, 1), Wo3, bo3.reshape(1, 3),
      Wa3, ba3.reshape(1, 1), w.reshape(B, 1, 3))

    return (w, sdf, img, aud)


# speedup vs baseline: 1.3882x; 1.1247x over previous
import functools

import jax
import jax.numpy as jnp
from jax.experimental import pallas as pl
from jax.experimental.pallas import tpu as pltpu


def _fused_body(TN, H,
                ui_ref, W1_ref, b1_ref, W2_ref, b2_ref, Wr_ref, br_ref,
                Wt_ref, bt_ref,
                Wg1_ref, bg1_ref, Wo1_ref, bo1_ref, Wa1_ref, ba1_ref,
                Wg2_ref, bg2_ref, Wo2_ref, bo2_ref, Wa2_ref, ba2_ref,
                Wgs_ref, bgs_ref, Wo3_ref, bo3_ref, Wa3_ref, ba3_ref,
                c3_ref, c2_ref, c1_ref,
                w_ref, sdf_ref, img_ref, aud_ref,
                bias_s, wf_s, wg2_s, wo2_s, wa2_s):
    b = pl.program_id(0)
    n = pl.program_id(1)
    f32 = jnp.float32
    bf16 = jnp.bfloat16

    @pl.when((b == 0) & (n == 0))
    def _router():
        h = ui_ref[...]
        W1 = W1_ref[...]
        W2 = W2_ref[...]
        b1 = b1_ref[...]
        b2 = b2_ref[...]
        for _ in range(3):
            m = jnp.tanh(jnp.dot(h, W1, preferred_element_type=f32) + b1)
            h = h + jnp.tanh(jnp.dot(m, W2, preferred_element_type=f32) + b2)
        logits = jnp.dot(h, Wr_ref[...], preferred_element_type=f32) + br_ref[...]
        logits = logits - jnp.max(logits, axis=-1, keepdims=True)
        e = jnp.exp(logits)
        w_ref[...] = e / jnp.sum(e, axis=-1, keepdims=True)
        th = jnp.tanh(jnp.dot(h, Wt_ref[...], preferred_element_type=f32) + bt_ref[...])
        pg = jnp.dot(th, Wg1_ref[3:], preferred_element_type=f32) + bg1_ref[...]
        po = jnp.dot(th, Wo1_ref[2:], preferred_element_type=f32) + bo1_ref[...]
        pa = jnp.dot(th, Wa1_ref[1:], preferred_element_type=f32) + ba1_ref[...]
        bias_s[...] = jnp.concatenate([pg, po, pa], axis=1)
        z = lambda r: jnp.zeros((r, H), f32)
        wg = jnp.concatenate([Wg1_ref[:3], z(5)], axis=0)
        wo = jnp.concatenate([z(3), Wo1_ref[:2], z(3)], axis=0)
        wa = jnp.concatenate([z(5), Wa1_ref[:1], z(2)], axis=0)
        wf_s[...] = jnp.concatenate([wg, wo, wa], axis=1).astype(bf16)
        wg2_s[...] = Wg2_ref[...].astype(bf16)
        wo2_s[...] = Wo2_ref[...].astype(bf16)
        wa2_s[...] = Wa2_ref[...].astype(bf16)

    w = w_ref[pl.ds(b, 1), :]
    bias = bias_s[pl.ds(b, 1), :]
    pad = jnp.zeros((TN, 2), bf16)
    cat = jnp.concatenate(
        [c3_ref[0].astype(bf16), c2_ref[0].astype(bf16),
         c1_ref[0].astype(bf16), pad], axis=-1)
    h1 = jnp.dot(cat, wf_s[...], preferred_element_type=f32)
    h1 = jnp.maximum(h1 + bias, 0.0).astype(bf16)

    hg = jnp.dot(h1[:, :H], wg2_s[...],
                 preferred_element_type=f32) + bg2_ref[...]
    ho = jnp.dot(h1[:, H:2 * H], wo2_s[...],
                 preferred_element_type=f32) + bo2_ref[...]
    ha = jnp.dot(h1[:, 2 * H:], wa2_s[...],
                 preferred_element_type=f32) + ba2_ref[...]
    hg = jnp.maximum(hg, 0.0)
    ho = jnp.maximum(ho, 0.0)
    ha = jnp.maximum(ha, 0.0)

    sdf = jnp.dot(hg, Wgs_ref[...], preferred_element_type=f32)
    img = jnp.dot(ho, Wo3_ref[...], preferred_element_type=f32)
    aud = jnp.dot(ha, Wa3_ref[...], preferred_element_type=f32)
    sdf_ref[0] = (sdf + bgs_ref[...]) * w[0:1, 0:1]
    img_ref[0] = jax.nn.sigmoid(img + bo3_ref[...]) * w[0:1, 1:2]
    aud_ref[0] = jnp.tanh(aud + ba3_ref[...]) * w[0:1, 2:3]


@functools.partial(jax.jit, static_argnames=("interpret",))
def kernel(user_intent, coords_3d, coords_2d, coords_1d,
           W1, b1, W2, b2, Wr, br, Wt, bt,
           Wg1, bg1, Wg2, bg2, Wgs, bgs, Wgc, bgc,
           Wo1, bo1, Wo2, bo2, Wo3, bo3,
           Wa1, ba1, Wa2, ba2, Wa3, ba3, interpret=False):
    B, N, _ = coords_3d.shape
    GD = user_intent.shape[1]
    LD = Wt.shape[1]
    H = Wg2.shape[0]
    TN = 2048
    f32 = jnp.float32
    bf16 = jnp.bfloat16

    def const(shape):
        return pl.BlockSpec(shape, lambda b, n: tuple(0 for _ in shape))

    w, sdf, img, aud = pl.pallas_call(
        functools.partial(_fused_body, TN, H),
        grid=(B, N // TN),
        in_specs=[
            const((B, GD)),
            const((GD, GD)), const((1, GD)),
            const((GD, GD)), const((1, GD)),
            const((GD, 3)), const((1, 3)),
            const((GD, LD)), const((1, LD)),
            const((3 + LD, H)), const((1, H)),
            const((2 + LD, H)), const((1, H)),
            const((1 + LD, H)), const((1, H)),
            const((H, H)), const((1, H)),
            const((H, H)), const((1, H)),
            const((H, H)), const((1, H)),
            const((H, 1)), const((1, 1)),
            const((H, 3)), const((1, 3)),
            const((H, 1)), const((1, 1)),
            pl.BlockSpec((1, TN, 3), lambda b, n: (b, n, 0)),
            pl.BlockSpec((1, TN, 2), lambda b, n: (b, n, 0)),
            pl.BlockSpec((1, TN, 1), lambda b, n: (b, n, 0)),
        ],
        out_specs=[
            const((B, 3)),
            pl.BlockSpec((1, TN, 1), lambda b, n: (b, n, 0)),
            pl.BlockSpec((1, TN, 3), lambda b, n: (b, n, 0)),
            pl.BlockSpec((1, TN, 1), lambda b, n: (b, n, 0)),
        ],
        out_shape=(
            jax.ShapeDtypeStruct((B, 3), f32),
            jax.ShapeDtypeStruct((B, N, 1), f32),
            jax.ShapeDtypeStruct((B, N, 3), f32),
            jax.ShapeDtypeStruct((B, N, 1), f32),
        ),
        scratch_shapes=[
            pltpu.VMEM((B, 3 * H), f32),
            pltpu.VMEM((8, 3 * H), bf16),
            pltpu.VMEM((H, H), bf16),
            pltpu.VMEM((H, H), bf16),
            pltpu.VMEM((H, H), bf16),
        ],
        compiler_params=pltpu.CompilerParams(
            dimension_semantics=("arbitrary", "arbitrary")),
        interpret=interpret,
    )(user_intent, W1, b1.reshape(1, GD), W2, b2.reshape(1, GD),
      Wr, br.reshape(1, 3), Wt, bt.reshape(1, LD),
      Wg1, bg1.reshape(1, H), Wo1, bo1.reshape(1, H), Wa1, ba1.reshape(1, H),
      Wg2, bg2.reshape(1, H), Wo2, bo2.reshape(1, H), Wa2, ba2.reshape(1, H),
      Wgs, bgs.reshape(1, 1), Wo3, bo3.reshape(1, 3), Wa3, ba3.reshape(1, 1),
      coords_3d, coords_2d, coords_1d)

    return (w, sdf, img, aud)


# speedup vs baseline: 1.4209x; 1.0235x over previous
import functools

import jax
import jax.numpy as jnp
from jax.experimental import pallas as pl
from jax.experimental.pallas import tpu as pltpu


def _fused_body(TN, H,
                ui_ref, W1_ref, b1_ref, W2_ref, b2_ref, Wr_ref, br_ref,
                Wt_ref, bt_ref,
                Wg1_ref, bg1_ref, Wo1_ref, bo1_ref, Wa1_ref, ba1_ref,
                Wg2_ref, bg2_ref, Wo2_ref, bo2_ref, Wa2_ref, ba2_ref,
                Wgs_ref, bgs_ref, Wo3_ref, bo3_ref, Wa3_ref, ba3_ref,
                c3_ref, c2_ref, c1_ref,
                w_ref, sdf_ref, img_ref, aud_ref,
                bias_s, wf_s, wg2_s, wo2_s, wa2_s):
    b = pl.program_id(0)
    n = pl.program_id(1)
    f32 = jnp.float32
    bf16 = jnp.bfloat16

    @pl.when((b == 0) & (n == 0))
    def _router():
        h = ui_ref[...]
        W1 = W1_ref[...]
        W2 = W2_ref[...]
        b1 = b1_ref[...]
        b2 = b2_ref[...]
        for _ in range(3):
            m = jnp.tanh(jnp.dot(h, W1, preferred_element_type=f32) + b1)
            h = h + jnp.tanh(jnp.dot(m, W2, preferred_element_type=f32) + b2)
        logits = jnp.dot(h, Wr_ref[...], preferred_element_type=f32) + br_ref[...]
        logits = logits - jnp.max(logits, axis=-1, keepdims=True)
        e = jnp.exp(logits)
        w_ref[...] = e / jnp.sum(e, axis=-1, keepdims=True)
        th = jnp.tanh(jnp.dot(h, Wt_ref[...], preferred_element_type=f32) + bt_ref[...])
        pg = jnp.dot(th, Wg1_ref[3:], preferred_element_type=f32) + bg1_ref[...]
        po = jnp.dot(th, Wo1_ref[2:], preferred_element_type=f32) + bo1_ref[...]
        pa = jnp.dot(th, Wa1_ref[1:], preferred_element_type=f32) + ba1_ref[...]
        bias_s[...] = jnp.concatenate([pg, po, pa], axis=1)
        z = lambda r: jnp.zeros((r, H), f32)
        wg = jnp.concatenate([Wg1_ref[:3], z(5)], axis=0)
        wo = jnp.concatenate([z(3), Wo1_ref[:2], z(3)], axis=0)
        wa = jnp.concatenate([z(5), Wa1_ref[:1], z(2)], axis=0)
        wf_s[...] = jnp.concatenate([wg, wo, wa], axis=1).astype(bf16)
        wg2_s[...] = Wg2_ref[...].astype(bf16)
        wo2_s[...] = Wo2_ref[...].astype(bf16)
        wa2_s[...] = Wa2_ref[...].astype(bf16)

    w = w_ref[pl.ds(b, 1), :]
    bias = bias_s[pl.ds(b, 1), :]
    pad = jnp.zeros((TN, 2), bf16)
    cat = jnp.concatenate(
        [c3_ref[0].astype(bf16), c2_ref[0].astype(bf16),
         c1_ref[0].astype(bf16), pad], axis=-1)
    h1 = jnp.dot(cat, wf_s[...], preferred_element_type=f32)
    h1 = jnp.maximum(h1 + bias, 0.0).astype(bf16)

    hg = jnp.dot(h1[:, :H], wg2_s[...],
                 preferred_element_type=f32) + bg2_ref[...]
    ho = jnp.dot(h1[:, H:2 * H], wo2_s[...],
                 preferred_element_type=f32) + bo2_ref[...]
    ha = jnp.dot(h1[:, 2 * H:], wa2_s[...],
                 preferred_element_type=f32) + ba2_ref[...]
    hg = jnp.maximum(hg, 0.0).astype(bf16)
    ho = jnp.maximum(ho, 0.0).astype(bf16)
    ha = jnp.maximum(ha, 0.0).astype(bf16)

    sdf = jnp.dot(hg, Wgs_ref[...].astype(bf16), preferred_element_type=f32)
    img = jnp.dot(ho, Wo3_ref[...].astype(bf16), preferred_element_type=f32)
    aud = jnp.dot(ha, Wa3_ref[...].astype(bf16), preferred_element_type=f32)
    sdf_ref[0] = (sdf + bgs_ref[...]) * w[0:1, 0:1]
    img_ref[0] = jax.nn.sigmoid(img + bo3_ref[...]) * w[0:1, 1:2]
    aud_ref[0] = jnp.tanh(aud + ba3_ref[...]) * w[0:1, 2:3]


@functools.partial(jax.jit, static_argnames=("interpret",))
def kernel(user_intent, coords_3d, coords_2d, coords_1d,
           W1, b1, W2, b2, Wr, br, Wt, bt,
           Wg1, bg1, Wg2, bg2, Wgs, bgs, Wgc, bgc,
           Wo1, bo1, Wo2, bo2, Wo3, bo3,
           Wa1, ba1, Wa2, ba2, Wa3, ba3, interpret=False):
    B, N, _ = coords_3d.shape
    GD = user_intent.shape[1]
    LD = Wt.shape[1]
    H = Wg2.shape[0]
    TN = 4096
    f32 = jnp.float32
    bf16 = jnp.bfloat16

    def const(shape):
        return pl.BlockSpec(shape, lambda b, n: tuple(0 for _ in shape))

    w, sdf, img, aud = pl.pallas_call(
        functools.partial(_fused_body, TN, H),
        grid=(B, N // TN),
        in_specs=[
            const((B, GD)),
            const((GD, GD)), const((1, GD)),
            const((GD, GD)), const((1, GD)),
            const((GD, 3)), const((1, 3)),
            const((GD, LD)), const((1, LD)),
            const((3 + LD, H)), const((1, H)),
            const((2 + LD, H)), const((1, H)),
            const((1 + LD, H)), const((1, H)),
            const((H, H)), const((1, H)),
            const((H, H)), const((1, H)),
            const((H, H)), const((1, H)),
            const((H, 1)), const((1, 1)),
            const((H, 3)), const((1, 3)),
            const((H, 1)), const((1, 1)),
            pl.BlockSpec((1, TN, 3), lambda b, n: (b, n, 0)),
            pl.BlockSpec((1, TN, 2), lambda b, n: (b, n, 0)),
            pl.BlockSpec((1, TN, 1), lambda b, n: (b, n, 0)),
        ],
        out_specs=[
            const((B, 3)),
            pl.BlockSpec((1, TN, 1), lambda b, n: (b, n, 0)),
            pl.BlockSpec((1, TN, 3), lambda b, n: (b, n, 0)),
            pl.BlockSpec((1, TN, 1), lambda b, n: (b, n, 0)),
        ],
        out_shape=(
            jax.ShapeDtypeStruct((B, 3), f32),
            jax.ShapeDtypeStruct((B, N, 1), f32),
            jax.ShapeDtypeStruct((B, N, 3), f32),
            jax.ShapeDtypeStruct((B, N, 1), f32),
        ),
        scratch_shapes=[
            pltpu.VMEM((B, 3 * H), f32),
            pltpu.VMEM((8, 3 * H), bf16),
            pltpu.VMEM((H, H), bf16),
            pltpu.VMEM((H, H), bf16),
            pltpu.VMEM((H, H), bf16),
        ],
        compiler_params=pltpu.CompilerParams(
            dimension_semantics=("arbitrary", "arbitrary")),
        interpret=interpret,
    )(user_intent, W1, b1.reshape(1, GD), W2, b2.reshape(1, GD),
      Wr, br.reshape(1, 3), Wt, bt.reshape(1, LD),
      Wg1, bg1.reshape(1, H), Wo1, bo1.reshape(1, H), Wa1, ba1.reshape(1, H),
      Wg2, bg2.reshape(1, H), Wo2, bo2.reshape(1, H), Wa2, ba2.reshape(1, H),
      Wgs, bgs.reshape(1, 1), Wo3, bo3.reshape(1, 3), Wa3, ba3.reshape(1, 1),
      coords_3d, coords_2d, coords_1d)

    return (w, sdf, img, aud)


# speedup vs baseline: 1.4271x; 1.0044x over previous
import functools

import jax
import jax.numpy as jnp
from jax.experimental import pallas as pl
from jax.experimental.pallas import tpu as pltpu


def _router_body(H,
                 ui_ref, W1_ref, b1_ref, W2_ref, b2_ref, Wr_ref, br_ref,
                 Wt_ref, bt_ref,
                 Wg1_ref, bg1_ref, Wo1_ref, bo1_ref, Wa1_ref, ba1_ref,
                 Wg2_ref, Wo2_ref, Wa2_ref,
                 w_ref, bias_ref, wf_ref, wg2_ref, wo2_ref, wa2_ref):
    f32 = jnp.float32
    bf16 = jnp.bfloat16
    h = ui_ref[...]
    W1 = W1_ref[...]
    W2 = W2_ref[...]
    b1 = b1_ref[...]
    b2 = b2_ref[...]
    for _ in range(3):
        m = jnp.tanh(jnp.dot(h, W1, preferred_element_type=f32) + b1)
        h = h + jnp.tanh(jnp.dot(m, W2, preferred_element_type=f32) + b2)
    logits = jnp.dot(h, Wr_ref[...], preferred_element_type=f32) + br_ref[...]
    logits = logits - jnp.max(logits, axis=-1, keepdims=True)
    e = jnp.exp(logits)
    w_ref[...] = e / jnp.sum(e, axis=-1, keepdims=True)
    th = jnp.tanh(jnp.dot(h, Wt_ref[...], preferred_element_type=f32) + bt_ref[...])
    pg = jnp.dot(th, Wg1_ref[3:], preferred_element_type=f32) + bg1_ref[...]
    po = jnp.dot(th, Wo1_ref[2:], preferred_element_type=f32) + bo1_ref[...]
    pa = jnp.dot(th, Wa1_ref[1:], preferred_element_type=f32) + ba1_ref[...]
    bias_ref[...] = jnp.concatenate([pg, po, pa], axis=1)
    z = lambda r: jnp.zeros((r, H), f32)
    wg = jnp.concatenate([Wg1_ref[:3], z(5)], axis=0)
    wo = jnp.concatenate([z(3), Wo1_ref[:2], z(3)], axis=0)
    wa = jnp.concatenate([z(5), Wa1_ref[:1], z(2)], axis=0)
    wf_ref[...] = jnp.concatenate([wg, wo, wa], axis=1).astype(bf16)
    wg2_ref[...] = Wg2_ref[...].astype(bf16)
    wo2_ref[...] = Wo2_ref[...].astype(bf16)
    wa2_ref[...] = Wa2_ref[...].astype(bf16)


def _expert_body(TN, H,
                 w_ref, bias_ref, wf_ref, wg2_ref, wo2_ref, wa2_ref,
                 bg2_ref, bo2_ref, ba2_ref,
                 Wgs_ref, bgs_ref, Wo3_ref, bo3_ref, Wa3_ref, ba3_ref,
                 c3_ref, c2_ref, c1_ref,
                 sdf_ref, img_ref, aud_ref):
    f32 = jnp.float32
    bf16 = jnp.bfloat16
    b = pl.program_id(0)
    w = w_ref[pl.ds(b, 1), :]
    bias = bias_ref[pl.ds(b, 1), :]
    pad = jnp.zeros((TN, 2), bf16)
    cat = jnp.concatenate(
        [c3_ref[0].astype(bf16), c2_ref[0].astype(bf16),
         c1_ref[0].astype(bf16), pad], axis=-1)
    h1 = jnp.dot(cat, wf_ref[...], preferred_element_type=f32)
    h1 = jnp.maximum(h1 + bias, 0.0).astype(bf16)

    hg = jnp.dot(h1[:, :H], wg2_ref[...],
                 preferred_element_type=f32) + bg2_ref[...]
    ho = jnp.dot(h1[:, H:2 * H], wo2_ref[...],
                 preferred_element_type=f32) + bo2_ref[...]
    ha = jnp.dot(h1[:, 2 * H:], wa2_ref[...],
                 preferred_element_type=f32) + ba2_ref[...]
    hg = jnp.maximum(hg, 0.0).astype(bf16)
    ho = jnp.maximum(ho, 0.0).astype(bf16)
    ha = jnp.maximum(ha, 0.0).astype(bf16)

    sdf = jnp.dot(hg, Wgs_ref[...].astype(bf16), preferred_element_type=f32)
    img = jnp.dot(ho, Wo3_ref[...].astype(bf16), preferred_element_type=f32)
    aud = jnp.dot(ha, Wa3_ref[...].astype(bf16), preferred_element_type=f32)
    sdf_ref[0] = (sdf + bgs_ref[...]) * w[0:1, 0:1]
    img_ref[0] = jax.nn.sigmoid(img + bo3_ref[...]) * w[0:1, 1:2]
    aud_ref[0] = jnp.tanh(aud + ba3_ref[...]) * w[0:1, 2:3]


@functools.partial(jax.jit, static_argnames=("interpret",))
def kernel(user_intent, coords_3d, coords_2d, coords_1d,
           W1, b1, W2, b2, Wr, br, Wt, bt,
           Wg1, bg1, Wg2, bg2, Wgs, bgs, Wgc, bgc,
           Wo1, bo1, Wo2, bo2, Wo3, bo3,
           Wa1, ba1, Wa2, ba2, Wa3, ba3, interpret=False):
    B, N, _ = coords_3d.shape
    GD = user_intent.shape[1]
    LD = Wt.shape[1]
    H = Wg2.shape[0]
    TN = 4096
    f32 = jnp.float32
    bf16 = jnp.bfloat16

    def const1(shape):
        return pl.BlockSpec(shape, lambda: tuple(0 for _ in shape))

    w, bias768, wfirst, wg2b, wo2b, wa2b = pl.pallas_call(
        functools.partial(_router_body, H),
        grid=(),
        in_specs=[
            const1((B, GD)),
            const1((GD, GD)), const1((1, GD)),
            const1((GD, GD)), const1((1, GD)),
            const1((GD, 3)), const1((1, 3)),
            const1((GD, LD)), const1((1, LD)),
            const1((3 + LD, H)), const1((1, H)),
            const1((2 + LD, H)), const1((1, H)),
            const1((1 + LD, H)), const1((1, H)),
            const1((H, H)), const1((H, H)), const1((H, H)),
        ],
        out_specs=[
            const1((B, 3)), const1((B, 3 * H)), const1((8, 3 * H)),
            const1((H, H)), const1((H, H)), const1((H, H)),
        ],
        out_shape=(
            jax.ShapeDtypeStruct((B, 3), f32),
            jax.ShapeDtypeStruct((B, 3 * H), f32),
            jax.ShapeDtypeStruct((8, 3 * H), bf16),
            jax.ShapeDtypeStruct((H, H), bf16),
            jax.ShapeDtypeStruct((H, H), bf16),
            jax.ShapeDtypeStruct((H, H), bf16),
        ),
        interpret=interpret,
    )(user_intent, W1, b1.reshape(1, GD), W2, b2.reshape(1, GD),
      Wr, br.reshape(1, 3), Wt, bt.reshape(1, LD),
      Wg1, bg1.reshape(1, H), Wo1, bo1.reshape(1, H), Wa1, ba1.reshape(1, H),
      Wg2, Wo2, Wa2)

    def const(shape):
        return pl.BlockSpec(shape, lambda b: tuple(0 for _ in shape))

    sdf, img, aud = pl.pallas_call(
        functools.partial(_expert_body, TN, H),
        grid=(B,),
        in_specs=[
            const((B, 3)),
            const((B, 3 * H)),
            const((8, 3 * H)),
            const((H, H)), const((H, H)), const((H, H)),
            const((1, H)), const((1, H)), const((1, H)),
            const((H, 1)), const((1, 1)),
            const((H, 3)), const((1, 3)),
            const((H, 1)), const((1, 1)),
            pl.BlockSpec((1, TN, 3), lambda b: (b, 0, 0)),
            pl.BlockSpec((1, TN, 2), lambda b: (b, 0, 0)),
            pl.BlockSpec((1, TN, 1), lambda b: (b, 0, 0)),
        ],
        out_specs=[
            pl.BlockSpec((1, TN, 1), lambda b: (b, 0, 0)),
            pl.BlockSpec((1, TN, 3), lambda b: (b, 0, 0)),
            pl.BlockSpec((1, TN, 1), lambda b: (b, 0, 0)),
        ],
        out_shape=(
            jax.ShapeDtypeStruct((B, N, 1), f32),
            jax.ShapeDtypeStruct((B, N, 3), f32),
            jax.ShapeDtypeStruct((B, N, 1), f32),
        ),
        compiler_params=pltpu.CompilerParams(
            dimension_semantics=("parallel",)),
        interpret=interpret,
    )(w, bias768, wfirst, wg2b, wo2b, wa2b,
      bg2.reshape(1, H), bo2.reshape(1, H), ba2.reshape(1, H),
      Wgs, bgs.reshape(1, 1), Wo3, bo3.reshape(1, 3), Wa3, ba3.reshape(1, 1),
      coords_3d, coords_2d, coords_1d)

    return (w, sdf, img, aud)
